# Initial kernel scaffold; baseline (speedup 1.0000x reference)
import jax
import jax.numpy as jnp
from jax.experimental import pallas as pl


def kernel(sub, rel, edge_index, edge_type, init_embed, positional_embedding, W_pos, b_pos, init_rel, loop_rel, W_in, W_out, W_loop, W_rel, bias_conv):
    raise NotImplementedError("write your pallas kernel here")


# speedup vs baseline: 7.1939x; 7.1939x over previous
import functools

import jax
import jax.numpy as jnp
from jax import lax
from jax.experimental import pallas as pl
from jax.experimental.pallas import tpu as pltpu
from jax.experimental.pallas import tpu_sc as plsc

N = 10000
E = 160000
HALF = E // 2
D = 256
CD = 128
NPAD = 10240
K = 128
NC, NS, L = 2, 16, 16

_MESH = dict(core_axis_name="c", subcore_axis_name="s")
_SC_PARAMS = pltpu.CompilerParams(needs_layout_passes=False)


def _tile_sched(s):
    nb = jnp.where(s == 15, 25, 40)
    ebase = jnp.where(s == 15, 15 * 5120, s * 5120)
    return nb, ebase


def _sc1_body(ei0, ei1, et, dvio, c_both,
              zeros_v, ones_v, src_v, dst_v, et_v, w_v, flat_v,
              deg_v, dinv_v, dinv_t, deg_sh, dinv_sh, c_sh):
    c = lax.axis_index("c")
    s = lax.axis_index("s")
    zf = jnp.zeros((L,), jnp.float32)

    @pl.loop(0, 1000)
    def _(i):
        zeros_v[pl.ds(i * L, L)] = zf

    of = jnp.ones((L,), jnp.float32)

    @pl.loop(0, K // L)
    def _(i):
        ones_v[pl.ds(i * L, L)] = of

    pltpu.sync_copy(zeros_v.at[pl.ds(0, 640)], deg_sh.at[pl.ds(s * 640, 640)])

    @pl.loop(0, 5)
    def _(j):
        pltpu.sync_copy(zeros_v, c_sh.at[pl.ds(s * 80000 + j * 16000, 16000)])

    plsc.subcore_barrier()

    nb, ebase = _tile_sched(s)
    ebase = ebase + c * HALF

    @pl.loop(0, nb)
    def _(b):
        off = ebase + b * K
        pltpu.sync_copy(ei0.at[pl.ds(off, K)], src_v)
        pltpu.sync_copy(ones_v, deg_sh.at[src_v], add=True)

    plsc.subcore_barrier()

    nbase = s * 640
    pltpu.sync_copy(deg_sh.at[pl.ds(nbase, 640)], deg_v)

    @pl.loop(0, 40)
    def _(i):
        sl = pl.ds(i * L, L)
        x = deg_v[sl]
        y = jnp.full((L,), 0.7, jnp.float32)
        for t in (4.0, 16.0, 64.0, 256.0, 1024.0, 4096.0, 16384.0, 65536.0):
            y = y * jnp.where(x >= t, 0.5, 1.0)
        hx = x * 0.5
        for _ in range(6):
            y = y * (1.5 - (hx * y) * y)
        dinv_v[sl] = jnp.where(x > 0.0, y, 0.0)

    pltpu.sync_copy(dinv_v, dinv_sh.at[pl.ds(nbase, 640)])
    pltpu.sync_copy(dinv_v, dvio.at[pl.ds(c * NPAD + nbase, 640)])
    plsc.subcore_barrier()

    pltpu.sync_copy(dinv_sh, dinv_t)

    @pl.loop(0, nb)
    def _(b):
        off = ebase + b * K
        pltpu.sync_copy(ei0.at[pl.ds(off, K)], src_v)
        pltpu.sync_copy(ei1.at[pl.ds(off, K)], dst_v)
        pltpu.sync_copy(et.at[pl.ds(off, K)], et_v)

        @pl.loop(0, K // L)
        def _(j):
            sl = pl.ds(j * L, L)
            w_v[sl] = plsc.load_gather(dinv_t, [src_v[sl]])
            flat_v[sl] = dst_v[sl] * CD + et_v[sl]

        pltpu.sync_copy(w_v, c_sh.at[flat_v], add=True)

    plsc.subcore_barrier()

    @pl.loop(0, 5)
    def _(j):
        base = s * 80000 + j * 16000
        pltpu.sync_copy(c_sh.at[pl.ds(base, 16000)],
                        c_both.at[pl.ds(c * (N * CD) + base, 16000)])


_sc1 = functools.partial(
    pl.kernel,
    out_type=[
        jax.ShapeDtypeStruct((NC * NPAD,), jnp.float32),
        jax.ShapeDtypeStruct((NC * N * CD,), jnp.float32),
    ],
    mesh=plsc.VectorSubcoreMesh(**_MESH),
    compiler_params=_SC_PARAMS,
    scratch_types=[
        pltpu.VMEM((16000,), jnp.float32),
        pltpu.VMEM((K,), jnp.float32),
        pltpu.VMEM((K,), jnp.int32),
        pltpu.VMEM((K,), jnp.int32),
        pltpu.VMEM((K,), jnp.int32),
        pltpu.VMEM((K,), jnp.float32),
        pltpu.VMEM((K,), jnp.int32),
        pltpu.VMEM((640,), jnp.float32),
        pltpu.VMEM((640,), jnp.float32),
        pltpu.VMEM((NPAD,), jnp.float32),
        pltpu.VMEM_SHARED((NPAD,), jnp.float32),
        pltpu.VMEM_SHARED((NPAD,), jnp.float32),
        pltpu.VMEM_SHARED((N * CD,), jnp.float32),
    ],
)(_sc1_body)


def _sc2_body(ei0, ei1, yin2, yout2, a_out,
              zeros2_v, src_v, src2_v, dst_v, rows_v, sem, a_sh):
    c = lax.axis_index("c")
    s = lax.axis_index("s")
    zf = jnp.zeros((L,), jnp.float32)

    @pl.loop(0, 128)
    def _(r):
        @pl.loop(0, CD // L)
        def _(j):
            zeros2_v[r, pl.ds(j * L, L)] = zf

    nb, ebase0 = _tile_sched(s)
    row_off = c * N

    def run_dir(y_ref, dir_idx, dir_off):
        @pl.loop(0, 5)
        def _(k2):
            pltpu.sync_copy(zeros2_v, a_sh.at[pl.ds(s * 640 + k2 * 128, 128)])

        plsc.subcore_barrier()
        ebase = ebase0 + dir_off

        @pl.loop(0, nb)
        def _(b):
            off = ebase + b * K
            pltpu.sync_copy(ei0.at[pl.ds(off, K)], src_v)
            pltpu.sync_copy(ei1.at[pl.ds(off, K)], dst_v)

            @pl.loop(0, K // L)
            def _(j):
                sl = pl.ds(j * L, L)
                src2_v[sl] = src_v[sl] + row_off

            pltpu.async_copy(y_ref.at[src2_v], rows_v, sem).wait()
            pltpu.sync_copy(rows_v, a_sh.at[dst_v], add=True)

        plsc.subcore_barrier()

        @pl.loop(0, 5)
        def _(k2):
            sl = pl.ds(s * 640 + k2 * 128, 128)
            pltpu.sync_copy(a_sh.at[sl], a_out.at[dir_idx * NC + c, sl])

        plsc.subcore_barrier()

    run_dir(yin2, 0, 0)
    run_dir(yout2, 1, HALF)


_sc2 = functools.partial(
    pl.kernel,
    out_type=[jax.ShapeDtypeStruct((2 * NC, NPAD, CD), jnp.float32)],
    mesh=plsc.VectorSubcoreMesh(**_MESH),
    compiler_params=_SC_PARAMS,
    scratch_types=[
        pltpu.VMEM((128, CD), jnp.float32),
        pltpu.VMEM((K,), jnp.int32),
        pltpu.VMEM((K,), jnp.int32),
        pltpu.VMEM((K,), jnp.int32),
        pltpu.VMEM((K, CD), jnp.float32),
        pltpu.SemaphoreType.DMA,
        pltpu.VMEM_SHARED((NPAD, CD), jnp.float32),
    ],
)(_sc2_body)


def _sc3_body(xo, rtab, sub, rel, sub_out, rel_out, idx_v, rows_v, sem):
    c = lax.axis_index("c")
    s = lax.axis_index("s")
    base = (s * NC + c) * 32
    pltpu.sync_copy(sub.at[pl.ds(base, 32)], idx_v)
    pltpu.async_copy(xo.at[idx_v], rows_v, sem).wait()
    pltpu.sync_copy(rows_v, sub_out.at[pl.ds(base, 32)])
    pltpu.sync_copy(rel.at[pl.ds(base, 32)], idx_v)
    pltpu.async_copy(rtab.at[idx_v], rows_v, sem).wait()
    pltpu.sync_copy(rows_v, rel_out.at[pl.ds(base, 32)])


_sc3 = functools.partial(
    pl.kernel,
    out_type=[jax.ShapeDtypeStruct((1024, D), jnp.float32)] * 2,
    mesh=plsc.VectorSubcoreMesh(**_MESH),
    compiler_params=_SC_PARAMS,
    scratch_types=[
        pltpu.VMEM((32,), jnp.int32),
        pltpu.VMEM((32, D), jnp.float32),
        pltpu.SemaphoreType.DMA,
    ],
)(_sc3_body)


_BM = 1000


def _tc1_body(init_ref, pos_ref, wpos_ref, bpos_ref, relp_ref, win_ref,
              wout_ref, wloop_ref, wrel_ref, bias_ref, lrel_ref, dvi_ref,
              dvo_ref, ci_ref, co_ref,
              yin0_ref, yin1_ref, yout0_ref, yout1_ref, p_ref, r_ref):
    f32 = jnp.float32
    dn_t = (((1,), (1,)), ((), ()))
    dn = (((1,), (0,)), ((), ()))
    pos = lax.dot_general(pos_ref[...], wpos_ref[...], dn_t,
                          preferred_element_type=f32)
    x = init_ref[...] + pos + bpos_ref[...]
    xwi = lax.dot_general(x, win_ref[...], dn, preferred_element_type=f32)
    xwo = lax.dot_general(x, wout_ref[...], dn, preferred_element_type=f32)
    xwl = lax.dot_general(x, wloop_ref[...], dn, preferred_element_type=f32)
    dvi = dvi_ref[...]
    dvo = dvo_ref[...]
    yin = dvi * xwi
    yout = dvo * xwo
    yin0_ref[...] = yin[:, :CD]
    yin1_ref[...] = yin[:, CD:]
    yout0_ref[...] = yout[:, :CD]
    yout1_ref[...] = yout[:, CD:]
    relp = relp_ref[...]
    rwin = lax.dot_general(relp, win_ref[...], dn, preferred_element_type=f32)
    rwout = lax.dot_general(relp, wout_ref[...], dn, preferred_element_type=f32)
    corr = dvi * lax.dot_general(ci_ref[...], rwin, dn,
                                 preferred_element_type=f32)
    corr += dvo * lax.dot_general(co_ref[...], rwout, dn,
                                  preferred_element_type=f32)
    lr = lax.dot_general(lrel_ref[...], wloop_ref[...], dn,
                         preferred_element_type=f32)
    p_ref[...] = (xwl - lr - corr) * (1.0 / 3.0) + bias_ref[...]
    r_ref[...] = lax.dot_general(relp, wrel_ref[...], dn,
                                 preferred_element_type=f32)[:100, :]


def _tc1(init_embed, pos_emb, w_pos, b_pos, rel_pad, w_in, w_out, w_loop,
         w_rel, bias, loop_rel, dinv_in, dinv_out, c_in, c_out):
    full = lambda shp: pl.BlockSpec(shp, lambda i: (0, 0))
    blk = lambda shp: pl.BlockSpec(shp, lambda i: (i, 0))
    return pl.pallas_call(
        _tc1_body,
        grid=(N // _BM,),
        in_specs=[
            blk((_BM, D)), blk((_BM, D)), full((D, D)), full((1, D)),
            full((CD, D)), full((D, D)), full((D, D)), full((D, D)),
            full((D, D)), full((1, D)), full((1, D)), blk((_BM, 1)),
            blk((_BM, 1)), blk((_BM, CD)), blk((_BM, CD)),
        ],
        out_specs=[
            blk((_BM, CD)), blk((_BM, CD)), blk((_BM, CD)), blk((_BM, CD)),
            blk((_BM, D)), full((100, D)),
        ],
        out_shape=[
            jax.ShapeDtypeStruct((N, CD), jnp.float32),
            jax.ShapeDtypeStruct((N, CD), jnp.float32),
            jax.ShapeDtypeStruct((N, CD), jnp.float32),
            jax.ShapeDtypeStruct((N, CD), jnp.float32),
            jax.ShapeDtypeStruct((N, D), jnp.float32),
            jax.ShapeDtypeStruct((100, D), jnp.float32),
        ],
    )(init_embed, pos_emb, w_pos, b_pos, rel_pad, w_in, w_out, w_loop,
      w_rel, bias, loop_rel, dinv_in, dinv_out, c_in, c_out)


def _tc2_body(p_ref, ain0_ref, ain1_ref, aout0_ref, aout1_ref, dvi_ref,
              dvo_ref, xo_ref):
    di = dvi_ref[...] * (1.0 / 3.0)
    do = dvo_ref[...] * (1.0 / 3.0)
    p = p_ref[...]
    lo = p[:, :CD] + di * ain0_ref[...] + do * aout0_ref[...]
    hi = p[:, CD:] + di * ain1_ref[...] + do * aout1_ref[...]
    xo_ref[...] = jnp.tanh(jnp.concatenate([lo, hi], axis=1))


def _tc2(p, ain0, ain1, aout0, aout1, dinv_in, dinv_out):
    blk = lambda shp: pl.BlockSpec(shp, lambda i: (i, 0))
    return pl.pallas_call(
        _tc2_body,
        grid=(N // _BM,),
        in_specs=[blk((_BM, D)), blk((_BM, CD)), blk((_BM, CD)),
                  blk((_BM, CD)), blk((_BM, CD)), blk((_BM, 1)),
                  blk((_BM, 1))],
        out_specs=blk((_BM, D)),
        out_shape=jax.ShapeDtypeStruct((N, D), jnp.float32),
    )(p, ain0, ain1, aout0, aout1, dinv_in, dinv_out)


def kernel(sub, rel, edge_index, edge_type, init_embed, positional_embedding,
           W_pos, b_pos, init_rel, loop_rel, W_in, W_out, W_loop, W_rel,
           bias_conv):
    ei = edge_index.astype(jnp.int32)
    et = edge_type.astype(jnp.int32)
    sub = sub.astype(jnp.int32)
    rel = rel.astype(jnp.int32)

    ei0 = ei[0]
    ei1 = ei[1]
    dvio, c_both = _sc1(ei0, ei1, et)
    dinv_in = dvio[:N].reshape(N, 1)
    dinv_out = dvio[NPAD:NPAD + N].reshape(N, 1)
    c_in = c_both[:N * CD].reshape(N, CD)
    c_out = c_both[N * CD:].reshape(N, CD)

    rel_pad = jnp.concatenate(
        [init_rel, loop_rel,
         jnp.zeros((CD - init_rel.shape[0] - 1, D), jnp.float32)], axis=0)

    yin0, yin1, yout0, yout1, p, r = _tc1(
        init_embed, positional_embedding, W_pos, b_pos.reshape(1, D),
        rel_pad, W_in, W_out, W_loop, W_rel, bias_conv.reshape(1, D),
        loop_rel, dinv_in, dinv_out, c_in, c_out)

    yin2 = jnp.concatenate([yin0, yin1], axis=0)
    yout2 = jnp.concatenate([yout0, yout1], axis=0)
    (a4,) = _sc2(ei0, ei1, yin2, yout2)
    ain0, ain1 = a4[0, :N], a4[1, :N]
    aout0, aout1 = a4[2, :N], a4[3, :N]

    x_out = _tc2(p, ain0, ain1, aout0, aout1, dinv_in, dinv_out)

    sub_emb, rel_emb = _sc3(x_out, r, sub, rel)
    return (sub_emb, rel_emb, x_out)


# speedup vs baseline: 13.4159x; 1.8649x over previous
import functools

import jax
import jax.numpy as jnp
from jax import lax
from jax.experimental import pallas as pl
from jax.experimental.pallas import tpu as pltpu
from jax.experimental.pallas import tpu_sc as plsc

N = 10000
E = 160000
HALF = E // 2
D = 256
CD = 128
NPAD = 10240
K = 128
NB = 40
NC, NS, L = 2, 16, 16

_MESH = dict(core_axis_name="c", subcore_axis_name="s")
_SC_PARAMS = pltpu.CompilerParams(needs_layout_passes=False)


def _tile_sched(s):
    nb = jnp.where(s == 15, 25, NB)
    ebase = jnp.where(s == 15, 15 * 5120, s * 5120)
    return nb, ebase


def _fire_loads(hbm_ref, ebase, nb, dst2d, sem):
    for b in range(NB):
        @pl.when(b < nb)
        def _():
            pltpu.async_copy(hbm_ref.at[pl.ds(ebase + b * K, K)],
                             dst2d.at[b], sem)


def _drain_loads(hbm_ref, ebase, nb, dst2d, sem):
    for b in range(NB):
        @pl.when(b < nb)
        def _():
            pltpu.make_async_copy(hbm_ref.at[pl.ds(ebase + b * K, K)],
                                  dst2d.at[b], sem).wait()


def _sc1_body(ei0, ei1, et, dvio, c_both,
              zeros_v, ones_v, srcd, dstd, etd, wd,
              deg_v, dinv_v, dinv_t, sem_ld, sem_st,
              deg_sh, dinv_sh, c_sh):
    c = lax.axis_index("c")
    s = lax.axis_index("s")
    zf = jnp.zeros((L,), jnp.float32)

    @pl.loop(0, 250)
    def _(i):
        zeros_v[pl.ds(i * L, L)] = zf

    of = jnp.ones((L,), jnp.float32)

    @pl.loop(0, K // L)
    def _(i):
        ones_v[pl.ds(i * L, L)] = of

    pltpu.sync_copy(zeros_v.at[pl.ds(0, 640)], deg_sh.at[pl.ds(s * 640, 640)])

    @pl.loop(0, 20)
    def _(j):
        pltpu.sync_copy(zeros_v, c_sh.at[pl.ds(s * 80000 + j * 4000, 4000)])

    nb, ebase = _tile_sched(s)
    ebase = ebase + c * HALF

    _fire_loads(ei0, ebase, nb, srcd, sem_ld)
    _fire_loads(ei1, ebase, nb, dstd, sem_ld)
    _fire_loads(et, ebase, nb, etd, sem_ld)
    _drain_loads(ei0, ebase, nb, srcd, sem_ld)
    _drain_loads(ei1, ebase, nb, dstd, sem_ld)
    _drain_loads(et, ebase, nb, etd, sem_ld)
    plsc.subcore_barrier()

    for b in range(NB):
        @pl.when(b < nb)
        def _():
            pltpu.async_copy(ones_v, deg_sh.at[srcd.at[b]], sem_st, add=True)
    for b in range(NB):
        @pl.when(b < nb)
        def _():
            pltpu.make_async_copy(ones_v, deg_sh.at[srcd.at[b]],
                                  sem_st).wait()

    plsc.subcore_barrier()

    nbase = s * 640
    pltpu.sync_copy(deg_sh.at[pl.ds(nbase, 640)], deg_v)

    @pl.loop(0, 40)
    def _(i):
        sl = pl.ds(i * L, L)
        x = deg_v[sl]
        y = jnp.full((L,), 0.7, jnp.float32)
        for t in (4.0, 16.0, 64.0, 256.0, 1024.0, 4096.0, 16384.0, 65536.0):
            y = y * jnp.where(x >= t, 0.5, 1.0)
        hx = x * 0.5
        for _ in range(6):
            y = y * (1.5 - (hx * y) * y)
        dinv_v[sl] = jnp.where(x > 0.0, y, 0.0)

    pltpu.sync_copy(dinv_v, dinv_sh.at[pl.ds(nbase, 640)])
    pltpu.sync_copy(dinv_v, dvio.at[pl.ds(c * NPAD + nbase, 640)])
    plsc.subcore_barrier()

    pltpu.sync_copy(dinv_sh, dinv_t)

    @pl.loop(0, nb)
    def _(b):
        @pl.loop(0, K // L)
        def _(j):
            sl = pl.ds(j * L, L)
            wd[b, sl] = plsc.load_gather(dinv_t, [srcd[b, sl]])
            dstd[b, sl] = dstd[b, sl] * CD + etd[b, sl]

    for b in range(NB):
        @pl.when(b < nb)
        def _():
            pltpu.async_copy(wd.at[b], c_sh.at[dstd.at[b]], sem_st, add=True)
    for b in range(NB):
        @pl.when(b < nb)
        def _():
            pltpu.make_async_copy(wd.at[b], c_sh.at[dstd.at[b]],
                                  sem_st).wait()

    plsc.subcore_barrier()

    @pl.loop(0, 5)
    def _(j):
        base = s * 80000 + j * 16000
        pltpu.sync_copy(c_sh.at[pl.ds(base, 16000)],
                        c_both.at[pl.ds(c * (N * CD) + base, 16000)])


_SC1_SCRATCH = [
    pltpu.VMEM((4000,), jnp.float32),
    pltpu.VMEM((K,), jnp.float32),
    pltpu.VMEM((NB, K), jnp.int32),
    pltpu.VMEM((NB, K), jnp.int32),
    pltpu.VMEM((NB, K), jnp.int32),
    pltpu.VMEM((NB, K), jnp.float32),
    pltpu.VMEM((640,), jnp.float32),
    pltpu.VMEM((640,), jnp.float32),
    pltpu.VMEM((NPAD,), jnp.float32),
    pltpu.SemaphoreType.DMA,
    pltpu.SemaphoreType.DMA,
    pltpu.VMEM_SHARED((NPAD,), jnp.float32),
    pltpu.VMEM_SHARED((NPAD,), jnp.float32),
    pltpu.VMEM_SHARED((N * CD,), jnp.float32),
]


_sc1 = functools.partial(
    pl.kernel,
    out_type=[
        jax.ShapeDtypeStruct((NC * NPAD,), jnp.float32),
        jax.ShapeDtypeStruct((NC * N * CD,), jnp.float32),
    ],
    mesh=plsc.VectorSubcoreMesh(**_MESH),
    compiler_params=_SC_PARAMS,
    scratch_types=_SC1_SCRATCH,
)(_sc1_body)


def _sc2_body(ei0, ei1, yin2, yout2, a_out,
              zeros2_v, srcd, dstd, rows0, rows1, sem_ld, sem_g0, sem_g1,
              a_sh):
    c = lax.axis_index("c")
    s = lax.axis_index("s")
    zf = jnp.zeros((L,), jnp.float32)

    @pl.loop(0, 32)
    def _(r):
        @pl.loop(0, CD // L)
        def _(j):
            zeros2_v[r, pl.ds(j * L, L)] = zf

    nb, ebase0 = _tile_sched(s)
    row_off = c * N

    def run_dir(y_ref, dir_idx, dir_off):
        @pl.loop(0, 20)
        def _(k2):
            pltpu.sync_copy(zeros2_v, a_sh.at[pl.ds(s * 640 + k2 * 32, 32)])

        ebase = ebase0 + dir_off
        _fire_loads(ei0, ebase, nb, srcd, sem_ld)
        _fire_loads(ei1, ebase, nb, dstd, sem_ld)
        _drain_loads(ei0, ebase, nb, srcd, sem_ld)
        _drain_loads(ei1, ebase, nb, dstd, sem_ld)

        @pl.loop(0, nb)
        def _(b):
            @pl.loop(0, K // L)
            def _(j):
                sl = pl.ds(j * L, L)
                srcd[b, sl] = srcd[b, sl] + row_off

        plsc.subcore_barrier()

        pltpu.async_copy(y_ref.at[srcd.at[0]], rows0, sem_g0)

        @pl.loop(0, nb)
        def _(b):
            @pl.when(b % 2 == 0)
            def _():
                pltpu.make_async_copy(y_ref.at[srcd.at[b]], rows0,
                                      sem_g0).wait()

                @pl.when(b + 1 < nb)
                def _():
                    pltpu.async_copy(y_ref.at[srcd.at[b + 1]], rows1, sem_g1)

                pltpu.sync_copy(rows0, a_sh.at[dstd.at[b]], add=True)

            @pl.when(b % 2 == 1)
            def _():
                pltpu.make_async_copy(y_ref.at[srcd.at[b]], rows1,
                                      sem_g1).wait()

                @pl.when(b + 1 < nb)
                def _():
                    pltpu.async_copy(y_ref.at[srcd.at[b + 1]], rows0, sem_g0)

                pltpu.sync_copy(rows1, a_sh.at[dstd.at[b]], add=True)

        plsc.subcore_barrier()

        @pl.loop(0, 5)
        def _(k2):
            sl = pl.ds(s * 640 + k2 * 128, 128)
            pltpu.sync_copy(a_sh.at[sl], a_out.at[dir_idx * NC + c, sl])

        plsc.subcore_barrier()

    run_dir(yin2, 0, 0)
    run_dir(yout2, 1, HALF)


_sc2 = functools.partial(
    pl.kernel,
    out_type=[jax.ShapeDtypeStruct((2 * NC, NPAD, CD), jnp.float32)],
    mesh=plsc.VectorSubcoreMesh(**_MESH),
    compiler_params=_SC_PARAMS,
    scratch_types=[
        pltpu.VMEM((32, CD), jnp.float32),
        pltpu.VMEM((NB, K), jnp.int32),
        pltpu.VMEM((NB, K), jnp.int32),
        pltpu.VMEM((K, CD), jnp.float32),
        pltpu.VMEM((K, CD), jnp.float32),
        pltpu.SemaphoreType.DMA,
        pltpu.SemaphoreType.DMA,
        pltpu.SemaphoreType.DMA,
        pltpu.VMEM_SHARED((NPAD, CD), jnp.float32),
    ],
)(_sc2_body)


def _sc3_body(xo, rtab, sub, rel, sub_out, rel_out, idx_v, rows_v, sem):
    c = lax.axis_index("c")
    s = lax.axis_index("s")
    base = (s * NC + c) * 32
    pltpu.sync_copy(sub.at[pl.ds(base, 32)], idx_v)
    pltpu.async_copy(xo.at[idx_v], rows_v, sem).wait()
    pltpu.sync_copy(rows_v, sub_out.at[pl.ds(base, 32)])
    pltpu.sync_copy(rel.at[pl.ds(base, 32)], idx_v)
    pltpu.async_copy(rtab.at[idx_v], rows_v, sem).wait()
    pltpu.sync_copy(rows_v, rel_out.at[pl.ds(base, 32)])


_sc3 = functools.partial(
    pl.kernel,
    out_type=[jax.ShapeDtypeStruct((1024, D), jnp.float32)] * 2,
    mesh=plsc.VectorSubcoreMesh(**_MESH),
    compiler_params=_SC_PARAMS,
    scratch_types=[
        pltpu.VMEM((32,), jnp.int32),
        pltpu.VMEM((32, D), jnp.float32),
        pltpu.SemaphoreType.DMA,
    ],
)(_sc3_body)


_BM = 1000


def _tc1_body(init_ref, pos_ref, wpos_ref, bpos_ref, relp_ref, win_ref,
              wout_ref, wloop_ref, wrel_ref, bias_ref, lrel_ref, dvi_ref,
              dvo_ref, ci_ref, co_ref,
              yin_ref, yout_ref, p_ref, r_ref):
    f32 = jnp.float32
    dn_t = (((1,), (1,)), ((), ()))
    dn = (((1,), (0,)), ((), ()))
    pos = lax.dot_general(pos_ref[...], wpos_ref[...], dn_t,
                          preferred_element_type=f32)
    x = init_ref[...] + pos + bpos_ref[...]
    xwi = lax.dot_general(x, win_ref[...], dn, preferred_element_type=f32)
    xwo = lax.dot_general(x, wout_ref[...], dn, preferred_element_type=f32)
    xwl = lax.dot_general(x, wloop_ref[...], dn, preferred_element_type=f32)
    dvi = dvi_ref[0]
    dvo = dvo_ref[0]
    yin = dvi * xwi
    yout = dvo * xwo
    yin_ref[0] = yin[:, :CD]
    yin_ref[1] = yin[:, CD:]
    yout_ref[0] = yout[:, :CD]
    yout_ref[1] = yout[:, CD:]
    relp = relp_ref[...]
    rwin = lax.dot_general(relp, win_ref[...], dn, preferred_element_type=f32)
    rwout = lax.dot_general(relp, wout_ref[...], dn, preferred_element_type=f32)
    corr = dvi * lax.dot_general(ci_ref[0], rwin, dn,
                                 preferred_element_type=f32)
    corr += dvo * lax.dot_general(co_ref[0], rwout, dn,
                                  preferred_element_type=f32)
    lr = lax.dot_general(lrel_ref[...], wloop_ref[...], dn,
                         preferred_element_type=f32)
    p_ref[...] = (xwl - lr - corr) * (1.0 / 3.0) + bias_ref[...]
    r_ref[...] = lax.dot_general(relp, wrel_ref[...], dn,
                                 preferred_element_type=f32)[:100, :]


def _tc1(init_embed, pos_emb, w_pos, b_pos, rel_pad, w_in, w_out, w_loop,
         w_rel, bias, loop_rel, dvio3, c3):
    full = lambda shp: pl.BlockSpec(shp, lambda i: (0, 0))
    blk = lambda shp: pl.BlockSpec(shp, lambda i: (i, 0))
    half3 = lambda shp, h: pl.BlockSpec(shp, lambda i, _h=h: (_h, i, 0))
    return pl.pallas_call(
        _tc1_body,
        grid=(N // _BM,),
        in_specs=[
            blk((_BM, D)), blk((_BM, D)), full((D, D)), full((1, D)),
            full((CD, D)), full((D, D)), full((D, D)), full((D, D)),
            full((D, D)), full((1, D)), full((1, D)),
            half3((1, _BM, 1), 0), half3((1, _BM, 1), 1),
            half3((1, _BM, CD), 0), half3((1, _BM, CD), 1),
        ],
        out_specs=[
            pl.BlockSpec((2, _BM, CD), lambda i: (0, i, 0)),
            pl.BlockSpec((2, _BM, CD), lambda i: (0, i, 0)),
            blk((_BM, D)), full((100, D)),
        ],
        out_shape=[
            jax.ShapeDtypeStruct((2, N, CD), jnp.float32),
            jax.ShapeDtypeStruct((2, N, CD), jnp.float32),
            jax.ShapeDtypeStruct((N, D), jnp.float32),
            jax.ShapeDtypeStruct((100, D), jnp.float32),
        ],
    )(init_embed, pos_emb, w_pos, b_pos, rel_pad, w_in, w_out, w_loop,
      w_rel, bias, loop_rel, dvio3, dvio3, c3, c3)


def _tc2_body(p_ref, ain0_ref, ain1_ref, aout0_ref, aout1_ref, dvi_ref,
              dvo_ref, xo_ref):
    di = dvi_ref[0] * (1.0 / 3.0)
    do = dvo_ref[0] * (1.0 / 3.0)
    p = p_ref[...]
    lo = p[:, :CD] + di * ain0_ref[0] + do * aout0_ref[0]
    hi = p[:, CD:] + di * ain1_ref[0] + do * aout1_ref[0]
    xo_ref[...] = jnp.tanh(jnp.concatenate([lo, hi], axis=1))


def _tc2(p, a4, dvio3):
    blk = lambda shp: pl.BlockSpec(shp, lambda i: (i, 0))
    half3 = lambda shp, h: pl.BlockSpec(shp, lambda i, _h=h: (_h, i, 0))
    return pl.pallas_call(
        _tc2_body,
        grid=(N // _BM,),
        in_specs=[blk((_BM, D)),
                  half3((1, _BM, CD), 0), half3((1, _BM, CD), 1),
                  half3((1, _BM, CD), 2), half3((1, _BM, CD), 3),
                  half3((1, _BM, 1), 0), half3((1, _BM, 1), 1)],
        out_specs=blk((_BM, D)),
        out_shape=jax.ShapeDtypeStruct((N, D), jnp.float32),
    )(p, a4, a4, a4, a4, dvio3, dvio3)


def kernel(sub, rel, edge_index, edge_type, init_embed, positional_embedding,
           W_pos, b_pos, init_rel, loop_rel, W_in, W_out, W_loop, W_rel,
           bias_conv):
    ei = edge_index.astype(jnp.int32)
    et = edge_type.astype(jnp.int32)
    sub = sub.astype(jnp.int32)
    rel = rel.astype(jnp.int32)

    ei0 = ei[0]
    ei1 = ei[1]
    dvio, c_both = _sc1(ei0, ei1, et)
    dvio3 = dvio.reshape(NC, NPAD, 1)
    c3 = c_both.reshape(NC, N, CD)

    rel_pad = jnp.concatenate(
        [init_rel, loop_rel,
         jnp.zeros((CD - init_rel.shape[0] - 1, D), jnp.float32)], axis=0)

    yin3, yout3, p, r = _tc1(
        init_embed, positional_embedding, W_pos, b_pos.reshape(1, D),
        rel_pad, W_in, W_out, W_loop, W_rel, bias_conv.reshape(1, D),
        loop_rel, dvio3, c3)

    yin2 = yin3.reshape(2 * N, CD)
    yout2 = yout3.reshape(2 * N, CD)
    (a4,) = _sc2(ei0, ei1, yin2, yout2)

    x_out = _tc2(p, a4, dvio3)

    sub_emb, rel_emb = _sc3(x_out, r, sub, rel)
    return (sub_emb, rel_emb, x_out)


# speedup vs baseline: 13.4611x; 1.0034x over previous
import functools

import jax
import jax.numpy as jnp
from jax import lax
from jax.experimental import pallas as pl
from jax.experimental.pallas import tpu as pltpu
from jax.experimental.pallas import tpu_sc as plsc

N = 10000
E = 160000
HALF = E // 2
D = 256
CD = 128
NPAD = 10240
K = 128
NB = 40
NC, NS, L = 2, 16, 16

_MESH = dict(core_axis_name="c", subcore_axis_name="s")
_SC_PARAMS = pltpu.CompilerParams(needs_layout_passes=False)


def _tile_sched(s):
    nb = jnp.where(s == 15, 25, NB)
    ebase = jnp.where(s == 15, 15 * 5120, s * 5120)
    return nb, ebase


def _fire_loads(hbm_ref, ebase, nb, dst2d, sem):
    for b in range(NB):
        @pl.when(b < nb)
        def _():
            pltpu.async_copy(hbm_ref.at[pl.ds(ebase + b * K, K)],
                             dst2d.at[b], sem)


def _drain_loads(hbm_ref, ebase, nb, dst2d, sem):
    for b in range(NB):
        @pl.when(b < nb)
        def _():
            pltpu.make_async_copy(hbm_ref.at[pl.ds(ebase + b * K, K)],
                                  dst2d.at[b], sem).wait()


def _sc1_body(ei0, ei1, et, dvio, c_both,
              zeros_v, ones_v, srcd, dstd, etd, wd,
              deg_v, dinv_v, dinv_t, sem_ld, sem_st,
              deg_sh, dinv_sh, c_sh):
    c = lax.axis_index("c")
    s = lax.axis_index("s")
    zf = jnp.zeros((L,), jnp.float32)

    @pl.loop(0, 250)
    def _(i):
        zeros_v[pl.ds(i * L, L)] = zf

    of = jnp.ones((L,), jnp.float32)

    @pl.loop(0, K // L)
    def _(i):
        ones_v[pl.ds(i * L, L)] = of

    pltpu.sync_copy(zeros_v.at[pl.ds(0, 640)], deg_sh.at[pl.ds(s * 640, 640)])

    @pl.loop(0, 20)
    def _(j):
        pltpu.sync_copy(zeros_v, c_sh.at[pl.ds(s * 80000 + j * 4000, 4000)])

    nb, ebase = _tile_sched(s)
    ebase = ebase + c * HALF

    _fire_loads(ei0, ebase, nb, srcd, sem_ld)
    _fire_loads(ei1, ebase, nb, dstd, sem_ld)
    _fire_loads(et, ebase, nb, etd, sem_ld)
    _drain_loads(ei0, ebase, nb, srcd, sem_ld)
    _drain_loads(ei1, ebase, nb, dstd, sem_ld)
    _drain_loads(et, ebase, nb, etd, sem_ld)
    plsc.subcore_barrier()

    for b in range(NB):
        @pl.when(b < nb)
        def _():
            pltpu.async_copy(ones_v, deg_sh.at[srcd.at[b]], sem_st, add=True)
    for b in range(NB):
        @pl.when(b < nb)
        def _():
            pltpu.make_async_copy(ones_v, deg_sh.at[srcd.at[b]],
                                  sem_st).wait()

    plsc.subcore_barrier()

    nbase = s * 640
    pltpu.sync_copy(deg_sh.at[pl.ds(nbase, 640)], deg_v)

    @pl.loop(0, 40)
    def _(i):
        sl = pl.ds(i * L, L)
        x = deg_v[sl]
        y = jnp.full((L,), 0.7, jnp.float32)
        for t in (4.0, 16.0, 64.0, 256.0, 1024.0, 4096.0, 16384.0, 65536.0):
            y = y * jnp.where(x >= t, 0.5, 1.0)
        hx = x * 0.5
        for _ in range(6):
            y = y * (1.5 - (hx * y) * y)
        dinv_v[sl] = jnp.where(x > 0.0, y, 0.0)

    pltpu.sync_copy(dinv_v, dinv_sh.at[pl.ds(nbase, 640)])
    pltpu.sync_copy(dinv_v, dvio.at[pl.ds(c * NPAD + nbase, 640)])
    plsc.subcore_barrier()

    pltpu.sync_copy(dinv_sh, dinv_t)

    @pl.loop(0, nb)
    def _(b):
        @pl.loop(0, K // L)
        def _(j):
            sl = pl.ds(j * L, L)
            wd[b, sl] = plsc.load_gather(dinv_t, [srcd[b, sl]])
            dstd[b, sl] = dstd[b, sl] * CD + etd[b, sl]

    for b in range(NB):
        @pl.when(b < nb)
        def _():
            pltpu.async_copy(wd.at[b], c_sh.at[dstd.at[b]], sem_st, add=True)
    for b in range(NB):
        @pl.when(b < nb)
        def _():
            pltpu.make_async_copy(wd.at[b], c_sh.at[dstd.at[b]],
                                  sem_st).wait()

    plsc.subcore_barrier()

    @pl.loop(0, 5)
    def _(j):
        base = s * 80000 + j * 16000
        pltpu.sync_copy(c_sh.at[pl.ds(base, 16000)],
                        c_both.at[pl.ds(c * (N * CD) + base, 16000)])


_SC1_SCRATCH = [
    pltpu.VMEM((4000,), jnp.float32),
    pltpu.VMEM((K,), jnp.float32),
    pltpu.VMEM((NB, K), jnp.int32),
    pltpu.VMEM((NB, K), jnp.int32),
    pltpu.VMEM((NB, K), jnp.int32),
    pltpu.VMEM((NB, K), jnp.float32),
    pltpu.VMEM((640,), jnp.float32),
    pltpu.VMEM((640,), jnp.float32),
    pltpu.VMEM((NPAD,), jnp.float32),
    pltpu.SemaphoreType.DMA,
    pltpu.SemaphoreType.DMA,
    pltpu.VMEM_SHARED((NPAD,), jnp.float32),
    pltpu.VMEM_SHARED((NPAD,), jnp.float32),
    pltpu.VMEM_SHARED((N * CD,), jnp.float32),
]


_sc1 = functools.partial(
    pl.kernel,
    out_type=[
        jax.ShapeDtypeStruct((NC * NPAD,), jnp.float32),
        jax.ShapeDtypeStruct((NC * N * CD,), jnp.float32),
    ],
    mesh=plsc.VectorSubcoreMesh(**_MESH),
    compiler_params=_SC_PARAMS,
    scratch_types=_SC1_SCRATCH,
)(_sc1_body)


def _sc2_body(ei0, ei1, yin2, yout2, a_out,
              zeros2_v, srcd, dstd, rows0, rows1, sem_ld, sem_g0, sem_g1,
              a_sh):
    c = lax.axis_index("c")
    s = lax.axis_index("s")
    zf = jnp.zeros((L,), jnp.float32)

    @pl.loop(0, 32)
    def _(r):
        @pl.loop(0, CD // L)
        def _(j):
            zeros2_v[r, pl.ds(j * L, L)] = zf

    nb, ebase0 = _tile_sched(s)
    row_off = c * N

    def run_dir(y_ref, dir_idx, dir_off):
        @pl.loop(0, 20)
        def _(k2):
            pltpu.sync_copy(zeros2_v, a_sh.at[pl.ds(s * 640 + k2 * 32, 32)])

        ebase = ebase0 + dir_off
        _fire_loads(ei0, ebase, nb, srcd, sem_ld)
        _fire_loads(ei1, ebase, nb, dstd, sem_ld)
        _drain_loads(ei0, ebase, nb, srcd, sem_ld)
        _drain_loads(ei1, ebase, nb, dstd, sem_ld)

        @pl.loop(0, nb)
        def _(b):
            @pl.loop(0, K // L)
            def _(j):
                sl = pl.ds(j * L, L)
                srcd[b, sl] = srcd[b, sl] + row_off

        plsc.subcore_barrier()

        pltpu.async_copy(y_ref.at[srcd.at[0]], rows0, sem_g0)

        @pl.loop(0, nb)
        def _(b):
            @pl.when(b % 2 == 0)
            def _():
                pltpu.make_async_copy(y_ref.at[srcd.at[b]], rows0,
                                      sem_g0).wait()

                @pl.when(b + 1 < nb)
                def _():
                    pltpu.async_copy(y_ref.at[srcd.at[b + 1]], rows1, sem_g1)

                pltpu.sync_copy(rows0, a_sh.at[dstd.at[b]], add=True)

            @pl.when(b % 2 == 1)
            def _():
                pltpu.make_async_copy(y_ref.at[srcd.at[b]], rows1,
                                      sem_g1).wait()

                @pl.when(b + 1 < nb)
                def _():
                    pltpu.async_copy(y_ref.at[srcd.at[b + 1]], rows0, sem_g0)

                pltpu.sync_copy(rows1, a_sh.at[dstd.at[b]], add=True)

        plsc.subcore_barrier()

        @pl.loop(0, 5)
        def _(k2):
            sl = pl.ds(s * 640 + k2 * 128, 128)
            pltpu.sync_copy(a_sh.at[sl], a_out.at[dir_idx * NC + c, sl])

        plsc.subcore_barrier()

    run_dir(yin2, 0, 0)
    run_dir(yout2, 1, HALF)


_sc2 = functools.partial(
    pl.kernel,
    out_type=[jax.ShapeDtypeStruct((2 * NC, NPAD, CD), jnp.float32)],
    mesh=plsc.VectorSubcoreMesh(**_MESH),
    compiler_params=_SC_PARAMS,
    scratch_types=[
        pltpu.VMEM((32, CD), jnp.float32),
        pltpu.VMEM((NB, K), jnp.int32),
        pltpu.VMEM((NB, K), jnp.int32),
        pltpu.VMEM((K, CD), jnp.float32),
        pltpu.VMEM((K, CD), jnp.float32),
        pltpu.SemaphoreType.DMA,
        pltpu.SemaphoreType.DMA,
        pltpu.SemaphoreType.DMA,
        pltpu.VMEM_SHARED((NPAD, CD), jnp.float32),
    ],
)(_sc2_body)


def _sc3_body(xo, rtab, sub, rel, sub_out, rel_out, idx_v, rows_v, sem):
    c = lax.axis_index("c")
    s = lax.axis_index("s")
    base = (s * NC + c) * 32
    pltpu.sync_copy(sub.at[pl.ds(base, 32)], idx_v)
    pltpu.async_copy(xo.at[idx_v], rows_v, sem).wait()
    pltpu.sync_copy(rows_v, sub_out.at[pl.ds(base, 32)])
    pltpu.sync_copy(rel.at[pl.ds(base, 32)], idx_v)
    pltpu.async_copy(rtab.at[idx_v], rows_v, sem).wait()
    pltpu.sync_copy(rows_v, rel_out.at[pl.ds(base, 32)])


_sc3 = functools.partial(
    pl.kernel,
    out_type=[jax.ShapeDtypeStruct((1024, D), jnp.float32)] * 2,
    mesh=plsc.VectorSubcoreMesh(**_MESH),
    compiler_params=_SC_PARAMS,
    scratch_types=[
        pltpu.VMEM((32,), jnp.int32),
        pltpu.VMEM((32, D), jnp.float32),
        pltpu.SemaphoreType.DMA,
    ],
)(_sc3_body)


_BM = 2000
_CB = _BM * CD


def _tc1_body(init_ref, pos_ref, wpos_ref, bpos_ref, relp_ref, w3_ref,
              w3b_ref, bias_ref, lrel_ref, dvi_ref,
              dvo_ref, ci_ref, co_ref,
              yin_ref, yout_ref, p_ref, r_ref):
    f32 = jnp.float32
    dn_t = (((1,), (1,)), ((), ()))
    dn = (((1,), (0,)), ((), ()))
    pos = lax.dot_general(pos_ref[...], wpos_ref[...], dn_t,
                          preferred_element_type=f32)
    x = init_ref[...] + pos + bpos_ref[...]
    xw3 = lax.dot_general(x, w3_ref[...], dn, preferred_element_type=f32)
    dvi = dvi_ref[0]
    dvo = dvo_ref[0]
    yin = dvi * xw3[:, :D]
    yout = dvo * xw3[:, D:2 * D]
    xwl = xw3[:, 2 * D:]
    yin_ref[0] = yin[:, :CD]
    yin_ref[1] = yin[:, CD:]
    yout_ref[0] = yout[:, :CD]
    yout_ref[1] = yout[:, CD:]
    rw3 = lax.dot_general(relp_ref[...], w3b_ref[...], dn,
                          preferred_element_type=f32)
    ci = ci_ref[...].reshape(_BM, CD)
    co = co_ref[...].reshape(_BM, CD)
    corr = dvi * lax.dot_general(ci, rw3[:, :D], dn,
                                 preferred_element_type=f32)
    corr += dvo * lax.dot_general(co, rw3[:, D:2 * D], dn,
                                  preferred_element_type=f32)
    lr = lax.dot_general(lrel_ref[...], w3_ref[...], dn,
                         preferred_element_type=f32)[:, 2 * D:]
    p_ref[...] = (xwl - lr - corr) * (1.0 / 3.0) + bias_ref[...]
    r_ref[...] = rw3[:100, 2 * D:]


def _tc1(init_embed, pos_emb, w_pos, b_pos, rel_pad, w3, w3b,
         bias, loop_rel, dvio3, c_flat):
    full = lambda shp: pl.BlockSpec(shp, lambda i: tuple(0 for _ in shp))
    blk = lambda shp: pl.BlockSpec(shp, lambda i: (i, 0))
    half3 = lambda shp, h: pl.BlockSpec(shp, lambda i, _h=h: (_h, i, 0))
    nblk = N // _BM
    return pl.pallas_call(
        _tc1_body,
        grid=(nblk,),
        in_specs=[
            blk((_BM, D)), blk((_BM, D)), full((D, D)), full((1, D)),
            full((CD, D)), full((D, 3 * D)), full((D, 3 * D)),
            full((1, D)), full((1, D)),
            half3((1, _BM, 1), 0), half3((1, _BM, 1), 1),
            pl.BlockSpec((_CB,), lambda i: (i,)),
            pl.BlockSpec((_CB,), lambda i, _n=nblk: (i + _n,)),
        ],
        out_specs=[
            pl.BlockSpec((2, _BM, CD), lambda i: (0, i, 0)),
            pl.BlockSpec((2, _BM, CD), lambda i: (0, i, 0)),
            blk((_BM, D)), full((100, D)),
        ],
        out_shape=[
            jax.ShapeDtypeStruct((2, N, CD), jnp.float32),
            jax.ShapeDtypeStruct((2, N, CD), jnp.float32),
            jax.ShapeDtypeStruct((N, D), jnp.float32),
            jax.ShapeDtypeStruct((100, D), jnp.float32),
        ],
    )(init_embed, pos_emb, w_pos, b_pos, rel_pad, w3, w3b,
      bias, loop_rel, dvio3, dvio3, c_flat, c_flat)


def _tc2_body(p_ref, ain0_ref, ain1_ref, aout0_ref, aout1_ref, dvi_ref,
              dvo_ref, xo_ref):
    di = dvi_ref[0] * (1.0 / 3.0)
    do = dvo_ref[0] * (1.0 / 3.0)
    p = p_ref[...]
    lo = p[:, :CD] + di * ain0_ref[0] + do * aout0_ref[0]
    hi = p[:, CD:] + di * ain1_ref[0] + do * aout1_ref[0]
    xo_ref[...] = jnp.tanh(jnp.concatenate([lo, hi], axis=1))


_BM2 = 2000


def _tc2(p, a4, dvio3):
    blk = lambda shp: pl.BlockSpec(shp, lambda i: (i, 0))
    half3 = lambda shp, h: pl.BlockSpec(shp, lambda i, _h=h: (_h, i, 0))
    return pl.pallas_call(
        _tc2_body,
        grid=(N // _BM2,),
        in_specs=[blk((_BM2, D)),
                  half3((1, _BM2, CD), 0), half3((1, _BM2, CD), 1),
                  half3((1, _BM2, CD), 2), half3((1, _BM2, CD), 3),
                  half3((1, _BM2, 1), 0), half3((1, _BM2, 1), 1)],
        out_specs=blk((_BM2, D)),
        out_shape=jax.ShapeDtypeStruct((N, D), jnp.float32),
    )(p, a4, a4, a4, a4, dvio3, dvio3)


def kernel(sub, rel, edge_index, edge_type, init_embed, positional_embedding,
           W_pos, b_pos, init_rel, loop_rel, W_in, W_out, W_loop, W_rel,
           bias_conv):
    ei = edge_index.astype(jnp.int32)
    et = edge_type.astype(jnp.int32)
    sub = sub.astype(jnp.int32)
    rel = rel.astype(jnp.int32)

    ei0 = ei[0]
    ei1 = ei[1]
    dvio, c_both = _sc1(ei0, ei1, et)
    dvio3 = dvio.reshape(NC, NPAD, 1)

    rel_pad = jnp.concatenate(
        [init_rel, loop_rel,
         jnp.zeros((CD - init_rel.shape[0] - 1, D), jnp.float32)], axis=0)
    w3 = jnp.concatenate([W_in, W_out, W_loop], axis=1)
    w3b = jnp.concatenate([W_in, W_out, W_rel], axis=1)

    yin3, yout3, p, r = _tc1(
        init_embed, positional_embedding, W_pos, b_pos.reshape(1, D),
        rel_pad, w3, w3b, bias_conv.reshape(1, D),
        loop_rel, dvio3, c_both)

    yin2 = yin3.reshape(2 * N, CD)
    yout2 = yout3.reshape(2 * N, CD)
    (a4,) = _sc2(ei0, ei1, yin2, yout2)

    x_out = _tc2(p, a4, dvio3)

    sub_emb, rel_emb = _sc3(x_out, r, sub, rel)
    return (sub_emb, rel_emb, x_out)
